# Initial kernel scaffold; baseline (speedup 1.0000x reference)
import jax
import jax.numpy as jnp
from jax import lax
from jax.experimental import pallas as pl
from jax.experimental.pallas import tpu as pltpu

_D = 768
_E = 64
_K = 8
_T = 32768
_BLK = 2048
_MIN32 = jnp.int32(-(2**31))


def _enc(v, lane):
    b = lax.bitcast_convert_type(v, jnp.int32)
    b = b ^ (lax.shift_right_arithmetic(b, 31) & jnp.int32(0x7FFFFFFF))
    return (b & jnp.int32(~63)) | (jnp.int32(63) - lane)


def _dec(m):
    b = m ^ (lax.shift_right_arithmetic(m, 31) & jnp.int32(0x7FFFFFFF))
    return lax.bitcast_convert_type(b, jnp.float32)


def _body(x_ref, wt_ref, idx_ref, scr_ref):
    x = x_ref[...]
    wt = wt_ref[...]
    logits = jnp.dot(x, wt, preferred_element_type=jnp.float32)
    lt = logits.T
    lane = lax.broadcasted_iota(jnp.int32, lt.shape, 0)
    enc = _enc(lt, lane)
    ms = []
    for k in range(_K):
        m = jnp.max(enc, axis=0, keepdims=True)
        ms.append(m)
        if k + 1 < _K:
            enc = jnp.where(enc == m, _MIN32, enc)
    mk = jnp.concatenate(ms, axis=0)
    idx_ref[...] = jnp.int32(63) - (mk & jnp.int32(63))
    vals = _dec(mk)
    e = jnp.exp(vals - vals[0:1])
    scr_ref[...] = e / jnp.sum(e, axis=0, keepdims=True)


def _router(inp, wt, interpret=False):
    return pl.pallas_call(
        _body,
        grid=(_T // _BLK,),
        in_specs=[
            pl.BlockSpec((_BLK, _D), lambda i: (i, 0)),
            pl.BlockSpec((_D, _E), lambda i: (0, 0)),
        ],
        out_specs=[
            pl.BlockSpec((_K, _BLK), lambda i: (0, i)),
            pl.BlockSpec((_K, _BLK), lambda i: (0, i)),
        ],
        out_shape=[
            jax.ShapeDtypeStruct((_K, _T), jnp.int32),
            jax.ShapeDtypeStruct((_K, _T), jnp.float32),
        ],
        compiler_params=pltpu.CompilerParams(
            dimension_semantics=("arbitrary",),
        ),
        interpret=interpret,
    )(inp, wt)


def kernel(inp, W):
    idx_t, scr_t = _router(inp, W.T)
    return (idx_t.T, scr_t.T)


# speedup vs baseline: 30.1598x; 30.1598x over previous
import jax
import jax.numpy as jnp
from jax import lax
from jax.experimental import pallas as pl
from jax.experimental.pallas import tpu as pltpu

_D = 768
_E = 64
_K = 8
_T = 32768
_BLK = 2048


def _enc(v, lane):
    b = lax.bitcast_convert_type(v, jnp.int32)
    b = b ^ (lax.shift_right_arithmetic(b, 31) & jnp.int32(0x7FFFFFFF))
    return (b & jnp.int32(~63)) | (jnp.int32(63) - lane)


def _dec(m):
    b = m ^ (lax.shift_right_arithmetic(m, 31) & jnp.int32(0x7FFFFFFF))
    return lax.bitcast_convert_type(b, jnp.float32)


def _body(x_ref, wt_ref, idx_ref, scr_ref):
    x = x_ref[...]
    wt = wt_ref[...]
    logits = jnp.dot(x, wt, preferred_element_type=jnp.float32)
    lt = logits.T
    lane = lax.broadcasted_iota(jnp.int32, lt.shape, 0)
    enc = _enc(lt, lane)
    ms = []
    for k in range(_K):
        m = jnp.max(enc, axis=0, keepdims=True)
        ms.append(m)
        if k + 1 < _K:
            enc = jnp.where(enc == m, jnp.int32(-(2**31)), enc)
    mk = jnp.concatenate(ms, axis=0)
    idx_ref[...] = jnp.int32(63) - (mk & jnp.int32(63))
    vals = _dec(mk)
    e = jnp.exp(vals - vals[0:1])
    scr_ref[...] = e / jnp.sum(e, axis=0, keepdims=True)


def _router(inp, wt, interpret=False):
    return pl.pallas_call(
        _body,
        grid=(_T // _BLK,),
        in_specs=[
            pl.BlockSpec((_BLK, _D), lambda i: (i, 0)),
            pl.BlockSpec((_D, _E), lambda i: (0, 0)),
        ],
        out_specs=[
            pl.BlockSpec((_K, _BLK), lambda i: (0, i)),
            pl.BlockSpec((_K, _BLK), lambda i: (0, i)),
        ],
        out_shape=[
            jax.ShapeDtypeStruct((_K, _T), jnp.int32),
            jax.ShapeDtypeStruct((_K, _T), jnp.float32),
        ],
        compiler_params=pltpu.CompilerParams(
            dimension_semantics=("arbitrary",),
        ),
        interpret=interpret,
    )(inp, wt)


def kernel(inp, W):
    idx_t, scr_t = _router(inp, W.T)
    return (idx_t.T, scr_t.T)


# speedup vs baseline: 32.8124x; 1.0880x over previous
import jax
import jax.numpy as jnp
from jax import lax
from jax.experimental import pallas as pl
from jax.experimental.pallas import tpu as pltpu

_D = 768
_E = 64
_K = 8
_T = 32768
_BLK = 4096


def _enc(v, lane):
    b = lax.bitcast_convert_type(v, jnp.int32)
    b = b ^ (lax.shift_right_arithmetic(b, 31) & jnp.int32(0x7FFFFFFF))
    return (b & jnp.int32(~63)) | (jnp.int32(63) - lane)


def _dec(m):
    b = m ^ (lax.shift_right_arithmetic(m, 31) & jnp.int32(0x7FFFFFFF))
    return lax.bitcast_convert_type(b, jnp.float32)


def _body(x_ref, wt_ref, idx_ref, scr_ref):
    x = x_ref[...]
    wt = wt_ref[...]
    logits = jnp.dot(x, wt, preferred_element_type=jnp.float32)
    lt = logits.T
    lane = lax.broadcasted_iota(jnp.int32, lt.shape, 0)
    enc = _enc(lt, lane)
    ms = []
    for k in range(_K):
        m = jnp.max(enc, axis=0, keepdims=True)
        ms.append(m)
        if k + 1 < _K:
            enc = jnp.where(enc == m, jnp.int32(-(2**31)), enc)
    mk = jnp.concatenate(ms, axis=0)
    idx_ref[...] = jnp.int32(63) - (mk & jnp.int32(63))
    vals = _dec(mk)
    e = jnp.exp(vals - vals[0:1])
    scr_ref[...] = e / jnp.sum(e, axis=0, keepdims=True)


def _router(inp, wt, interpret=False):
    return pl.pallas_call(
        _body,
        grid=(_T // _BLK,),
        in_specs=[
            pl.BlockSpec((_BLK, _D), lambda i: (i, 0)),
            pl.BlockSpec((_D, _E), lambda i: (0, 0)),
        ],
        out_specs=[
            pl.BlockSpec((_K, _BLK), lambda i: (0, i)),
            pl.BlockSpec((_K, _BLK), lambda i: (0, i)),
        ],
        out_shape=[
            jax.ShapeDtypeStruct((_K, _T), jnp.int32),
            jax.ShapeDtypeStruct((_K, _T), jnp.float32),
        ],
        compiler_params=pltpu.CompilerParams(
            dimension_semantics=("arbitrary",),
        ),
        interpret=interpret,
    )(inp, wt)


def kernel(inp, W):
    idx_t, scr_t = _router(inp, W.T)
    return (idx_t.T, scr_t.T)
